# baseline (speedup 1.0000x reference)
import functools

import jax
import jax.numpy as jnp
from jax import lax
from jax.experimental import pallas as pl
from jax.experimental.pallas import tpu as pltpu

_NPOINT = 1024
_K = 32


def _fps_kernel(x_ref, y_ref, z_ref, idx_ref, xs_ref, ys_ref, zs_ref, dist_ref):
    B, N = x_ref.shape
    CHUNK = 128
    x = x_ref[...]
    y = y_ref[...]
    z = z_ref[...]
    iota = lax.broadcasted_iota(jnp.int32, (B, N), 1)
    iota_c = lax.broadcasted_iota(jnp.int32, (B, CHUNK), 1)
    dist_ref[...] = jnp.full((B, N), 1e10, jnp.float32)

    def body(j, state):
        far, ai, ax, ay, az = state
        ohf = jnp.where(iota == far, 1.0, 0.0)
        cx = jnp.sum(x * ohf, axis=1, keepdims=True)
        cy = jnp.sum(y * ohf, axis=1, keepdims=True)
        cz = jnp.sum(z * ohf, axis=1, keepdims=True)
        msk = iota_c == j
        mski = jnp.where(msk, jnp.int32(1), jnp.int32(0))
        mskf = jnp.where(msk, 1.0, 0.0)
        ai = ai + far * mski
        ax = ax + cx * mskf
        ay = ay + cy * mskf
        az = az + cz * mskf
        dx = x - cx
        dy = y - cy
        dz = z - cz
        d = (dx * dx + dy * dy) + dz * dz
        dist = jnp.minimum(dist_ref[...], d)
        dist_ref[...] = dist
        m = jnp.max(dist, axis=1, keepdims=True)
        cand = jnp.where(dist == m, iota, jnp.int32(N))
        far = jnp.min(cand, axis=1, keepdims=True)
        return far, ai, ax, ay, az

    far = jnp.zeros((B, 1), jnp.int32)
    idx_ref[...] = jnp.zeros_like(idx_ref)
    xs_ref[...] = jnp.zeros_like(xs_ref)
    ys_ref[...] = jnp.zeros_like(ys_ref)
    zs_ref[...] = jnp.zeros_like(zs_ref)
    for o in range(_NPOINT // CHUNK):
        sl = slice(o * CHUNK, (o + 1) * CHUNK)
        far, ai, ax, ay, az = lax.fori_loop(
            0, CHUNK, body,
            (far, idx_ref[:, sl], xs_ref[:, sl], ys_ref[:, sl], zs_ref[:, sl]))
        idx_ref[:, sl] = ai
        xs_ref[:, sl] = ax
        ys_ref[:, sl] = ay
        zs_ref[:, sl] = az


def _run_fps(xyz, interpret=False):
    B, N, _ = xyz.shape
    x = xyz[..., 0]
    y = xyz[..., 1]
    z = xyz[..., 2]
    out_shapes = (
        jax.ShapeDtypeStruct((B, _NPOINT), jnp.int32),
        jax.ShapeDtypeStruct((B, _NPOINT), jnp.float32),
        jax.ShapeDtypeStruct((B, _NPOINT), jnp.float32),
        jax.ShapeDtypeStruct((B, _NPOINT), jnp.float32),
    )
    fps_idx, xs, ys, zs = pl.pallas_call(
        _fps_kernel,
        out_shape=out_shapes,
        scratch_shapes=[pltpu.VMEM((B, N), jnp.float32)],
        interpret=interpret,
    )(x, y, z)
    return fps_idx, jnp.stack([xs, ys, zs], axis=-1)


def _knn_kernel(q_ref, p_ref, idx_ref):
    S = q_ref.shape[1]
    N = p_ref.shape[2]
    H = N // 4
    K = _K
    q = q_ref[0]
    p = p_ref[0]
    d = lax.dot_general(q, p, dimension_numbers=(((1,), (0,)), ((), ())),
                        preferred_element_type=jnp.float32)
    qx = q[:, 0:1]
    qy = q[:, 1:2]
    qz = q[:, 2:3]
    d = d + ((qx * qx + qy * qy) + qz * qz) * 0.25
    px = p[0:1, :]
    py = p[1:2, :]
    pz = p[2:3, :]
    d = d + ((px * px + py * py) + pz * pz)
    iota2 = lax.broadcasted_iota(jnp.int32, (S, H), 1)
    zero_i = jnp.zeros((S, H), jnp.int32)
    vs = [d[:, i * H:(i + 1) * H] for i in range(4)]
    qs = [zero_i + jnp.int32(i) for i in range(4)]

    def cexch(a, b):
        va, qa = a
        vb, qb = b
        s = vb < va
        return ((jnp.minimum(va, vb), jnp.where(s, qb, qa)),
                (jnp.maximum(va, vb), jnp.where(s, qa, qb)))

    p0, p1, p2, p3 = (vs[0], qs[0]), (vs[1], qs[1]), (vs[2], qs[2]), (vs[3], qs[3])
    p0, p1 = cexch(p0, p1)
    p2, p3 = cexch(p2, p3)
    p0, p2 = cexch(p0, p2)
    p1, p3 = cexch(p1, p3)
    p1, p2 = cexch(p1, p2)
    f0, n1, n2, n3 = p0[0], p1[0], p2[0], p3[0]
    base4 = 4 * iota2
    t0 = base4 + p0[1]
    t1 = base4 + p1[1]
    t2 = base4 + p2[1]
    t3 = base4 + p3[1]
    iota_k = lax.broadcasted_iota(jnp.int32, (S, K), 1)
    idx_ref[...] = jnp.zeros_like(idx_ref)
    acc = idx_ref[0]
    big = jnp.float32(3.0e38)
    bigi = jnp.int32(N)
    two = jnp.int32(2)
    for k in range(K):
        m = jnp.min(f0, axis=1, keepdims=True)
        cand = jnp.where(f0 == m, t0, bigi)
        sel = jnp.min(cand, axis=1, keepdims=True)
        acc = acc + sel * jnp.where(iota_k == k, jnp.int32(1), jnp.int32(0))
        fl = lax.shift_right_logical(sel, two)
        u = iota2 == fl
        f0 = jnp.where(u, n1, f0)
        n1 = jnp.where(u, n2, n1)
        n2 = jnp.where(u, n3, n2)
        n3 = jnp.where(u, big, n3)
        t0 = jnp.where(u, t1, t0)
        t1 = jnp.where(u, t2, t1)
        t2 = jnp.where(u, t3, t2)
    idx_ref[0] = acc


def _run_knn(new_xyz, xyz, interpret=False):
    B, N, _ = xyz.shape
    S = new_xyz.shape[1]
    xyz_quads = xyz.reshape(B, N // 4, 4, 3)
    xyz_perm = jnp.concatenate([xyz_quads[:, :, i, :] for i in range(4)],
                               axis=1)
    xyzT = jnp.swapaxes(xyz_perm, 1, 2)
    return pl.pallas_call(
        _knn_kernel,
        grid=(B,),
        in_specs=[
            pl.BlockSpec((1, S, 3), lambda b: (b, 0, 0)),
            pl.BlockSpec((1, 3, N), lambda b: (b, 0, 0)),
        ],
        out_specs=pl.BlockSpec((1, S, _K), lambda b: (b, 0, 0)),
        out_shape=jax.ShapeDtypeStruct((B, S, _K), jnp.int32),
        interpret=interpret,
    )(-2.0 * new_xyz, xyzT)


def _group_kernel(points_hbm, idx_hbm, fps_hbm, out_hbm,
                  idx_v, idx_v2, cidx_a, cidx_b, rows_v, rows_v2, ctr_v,
                  out_v, out_v2, sem, sem_g2, sem_s1, sem_s2, sem2):
    NC = 2
    K = _K
    Q = 4
    QW = 256
    NCHUNK = QW // Q
    wid = lax.axis_index("s") * NC + lax.axis_index("c")
    base_val = (wid // 4) * 4096
    q0 = wid * QW

    pltpu.sync_copy(fps_hbm.at[pl.ds(q0, 128)], cidx_a)
    pltpu.sync_copy(fps_hbm.at[pl.ds(q0 + 128, 128)], cidx_b)

    def addbase_c(i, carry):
        sl = pl.ds(i * 16, 16)
        cidx_a[sl] = cidx_a[sl] + base_val
        cidx_b[sl] = cidx_b[sl] + base_val
        return carry

    lax.fori_loop(0, 8, addbase_c, 0)
    pltpu.async_copy(points_hbm.at[cidx_a], ctr_v.at[pl.ds(0, 128)], sem2).wait()
    pltpu.async_copy(points_hbm.at[cidx_b], ctr_v.at[pl.ds(128, 128)], sem2).wait()

    idx_b = (idx_v, idx_v2)
    rows_b = (rows_v, rows_v2)
    out_b = (out_v, out_v2)
    gs_b = (sem, sem_g2)
    ss_b = (sem_s1, sem_s2)

    def issue_gather(c, p):
        qbase = q0 + c * Q
        pltpu.sync_copy(idx_hbm.at[pl.ds(qbase * K, Q * K)], idx_b[p])

        def addbase(i, carry2):
            sl = pl.ds(i * 16, 16)
            idx_b[p][sl] = idx_b[p][sl] + base_val
            return carry2

        lax.fori_loop(0, Q * K // 16, addbase, 0)
        pltpu.make_async_copy(points_hbm.at[idx_b[p]], rows_b[p], gs_b[p]).start()

    def wait_gather(p):
        pltpu.make_async_copy(points_hbm.at[idx_b[p]], rows_b[p], gs_b[p]).wait()

    def issue_store(c, p):
        qbase = q0 + c * Q
        pltpu.make_async_copy(
            out_b[p], out_hbm.at[pl.ds(qbase * K, Q * K)], ss_b[p]).start()

    def wait_store(c, p):
        qbase = q0 + c * Q
        pltpu.make_async_copy(
            out_b[p], out_hbm.at[pl.ds(qbase * K, Q * K)], ss_b[p]).wait()

    def compute(c, p):
        def qbody(q, carry2):
            cvs = [ctr_v[c * Q + q, pl.ds(t * 16, 16)] for t in range(4)]

            def jbody(j, carry3):
                r = q * K + j
                for t in range(4):
                    g = rows_b[p][r, pl.ds(t * 16, 16)]
                    out_b[p][r, pl.ds(t * 16, 16)] = g - cvs[t]
                    out_b[p][r, pl.ds(64 + t * 16, 16)] = cvs[t]
                return carry3

            return lax.fori_loop(0, K, jbody, carry2)

        lax.fori_loop(0, Q, qbody, 0)

    issue_gather(0, 0)

    def outer_body(i, carry):
        for b in range(2):
            c = 2 * i + b

            @pl.when(c + 1 < NCHUNK)
            def _():
                issue_gather(c + 1, 1 - b)

            wait_gather(b)

            @pl.when(c >= 2)
            def _():
                wait_store(c - 2, b)

            compute(c, b)
            issue_store(c, b)
        return carry

    lax.fori_loop(0, NCHUNK // 2, outer_body, 0)
    wait_store(NCHUNK - 2, 0)
    wait_store(NCHUNK - 1, 1)


def _run_group(points, idx, fps_idx):
    B, N, D = points.shape
    S = idx.shape[1]
    K = _K
    Q = 4
    QW = 256
    points_flat = jnp.concatenate(
        [points, jnp.zeros_like(points)], axis=-1).reshape(B * N, 2 * D)
    idx_flat = idx.reshape(B * S * K)
    fps_flat = fps_idx.reshape(B * S)
    from jax.experimental.pallas import tpu_sc as plsc
    mesh = plsc.VectorSubcoreMesh(core_axis_name="c", subcore_axis_name="s")
    out = pl.kernel(
        _group_kernel,
        out_type=jax.ShapeDtypeStruct((B * S * K, 2 * D), jnp.float32),
        mesh=mesh,
        scratch_types=[
            pltpu.VMEM((Q * K,), jnp.int32),
            pltpu.VMEM((Q * K,), jnp.int32),
            pltpu.VMEM((128,), jnp.int32),
            pltpu.VMEM((128,), jnp.int32),
            pltpu.VMEM((Q * K, 2 * D), jnp.float32),
            pltpu.VMEM((Q * K, 2 * D), jnp.float32),
            pltpu.VMEM((QW, 2 * D), jnp.float32),
            pltpu.VMEM((Q * K, 2 * D), jnp.float32),
            pltpu.VMEM((Q * K, 2 * D), jnp.float32),
            pltpu.SemaphoreType.DMA,
            pltpu.SemaphoreType.DMA,
            pltpu.SemaphoreType.DMA,
            pltpu.SemaphoreType.DMA,
            pltpu.SemaphoreType.DMA,
        ],
    )(points_flat, idx_flat, fps_flat)
    return out.reshape(B, S, K, 2 * D)


def kernel(xyz, points):
    B, N, _ = xyz.shape
    D = points.shape[-1]
    fps_idx, new_xyz = _run_fps(xyz)

    idx = _run_knn(new_xyz, xyz)
    out = _run_group(points, idx, fps_idx)
    return (new_xyz, out)


# speedup vs baseline: 1.0208x; 1.0208x over previous
import functools

import jax
import jax.numpy as jnp
from jax import lax
from jax.experimental import pallas as pl
from jax.experimental.pallas import tpu as pltpu

_NPOINT = 1024
_K = 32


def _fps_kernel(x_ref, y_ref, z_ref, idx_ref, xs_ref, ys_ref, zs_ref, dist_ref):
    B, N = x_ref.shape
    CHUNK = 128
    x = x_ref[...]
    y = y_ref[...]
    z = z_ref[...]
    iota = lax.broadcasted_iota(jnp.int32, (B, N), 1)
    iota_c = lax.broadcasted_iota(jnp.int32, (B, CHUNK), 1)
    dist_ref[...] = jnp.full((B, N), 1e10, jnp.float32)

    def body(j, state):
        far, ai, ax, ay, az = state
        ohf = jnp.where(iota == far, 1.0, 0.0)
        cx = jnp.sum(x * ohf, axis=1, keepdims=True)
        cy = jnp.sum(y * ohf, axis=1, keepdims=True)
        cz = jnp.sum(z * ohf, axis=1, keepdims=True)
        msk = iota_c == j
        mski = jnp.where(msk, jnp.int32(1), jnp.int32(0))
        mskf = jnp.where(msk, 1.0, 0.0)
        ai = ai + far * mski
        ax = ax + cx * mskf
        ay = ay + cy * mskf
        az = az + cz * mskf
        dx = x - cx
        dy = y - cy
        dz = z - cz
        d = (dx * dx + dy * dy) + dz * dz
        dist = jnp.minimum(dist_ref[...], d)
        dist_ref[...] = dist
        m = jnp.max(dist, axis=1, keepdims=True)
        cand = jnp.where(dist == m, iota, jnp.int32(N))
        far = jnp.min(cand, axis=1, keepdims=True)
        return far, ai, ax, ay, az

    far = jnp.zeros((B, 1), jnp.int32)
    idx_ref[...] = jnp.zeros_like(idx_ref)
    xs_ref[...] = jnp.zeros_like(xs_ref)
    ys_ref[...] = jnp.zeros_like(ys_ref)
    zs_ref[...] = jnp.zeros_like(zs_ref)
    for o in range(_NPOINT // CHUNK):
        sl = slice(o * CHUNK, (o + 1) * CHUNK)
        far, ai, ax, ay, az = lax.fori_loop(
            0, CHUNK, body,
            (far, idx_ref[:, sl], xs_ref[:, sl], ys_ref[:, sl], zs_ref[:, sl]))
        idx_ref[:, sl] = ai
        xs_ref[:, sl] = ax
        ys_ref[:, sl] = ay
        zs_ref[:, sl] = az


def _run_fps(xyz, interpret=False):
    B, N, _ = xyz.shape
    x = xyz[..., 0]
    y = xyz[..., 1]
    z = xyz[..., 2]
    out_shapes = (
        jax.ShapeDtypeStruct((B, _NPOINT), jnp.int32),
        jax.ShapeDtypeStruct((B, _NPOINT), jnp.float32),
        jax.ShapeDtypeStruct((B, _NPOINT), jnp.float32),
        jax.ShapeDtypeStruct((B, _NPOINT), jnp.float32),
    )
    fps_idx, xs, ys, zs = pl.pallas_call(
        _fps_kernel,
        out_shape=out_shapes,
        scratch_shapes=[pltpu.VMEM((B, N), jnp.float32)],
        interpret=interpret,
    )(x, y, z)
    return fps_idx, jnp.stack([xs, ys, zs], axis=-1)


def _knn_kernel(q_ref, p_ref, idx_ref):
    S = q_ref.shape[1]
    N = p_ref.shape[2]
    H = N // 2
    K = _K
    q = q_ref[0]
    p = p_ref[0]
    d = lax.dot_general(q, p, dimension_numbers=(((1,), (0,)), ((), ())),
                        preferred_element_type=jnp.float32)
    qx = q[:, 0:1]
    qy = q[:, 1:2]
    qz = q[:, 2:3]
    d = d + ((qx * qx + qy * qy) + qz * qz) * 0.25
    px = p[0:1, :]
    py = p[1:2, :]
    pz = p[2:3, :]
    d = d + ((px * px + py * py) + pz * pz)
    dl = d[:, :H]
    dr = d[:, H:]
    side = jnp.where(dr < dl, jnp.int32(1), jnp.int32(0))
    fmin = jnp.minimum(dl, dr)
    nv = jnp.maximum(dl, dr)
    iota2 = lax.broadcasted_iota(jnp.int32, (S, H), 1)
    tiw = 2 * iota2 + side
    big = jnp.float32(3.0e38)
    bigi = jnp.int32(N)
    one = jnp.int32(1)
    for k in range(K):
        m = jnp.min(fmin, axis=1, keepdims=True)
        cand = jnp.where(fmin == m, tiw, bigi)
        sel = jnp.min(cand, axis=1, keepdims=True)
        idx_ref[0, :, k:k + 1] = sel
        fl = lax.shift_right_logical(sel, one)
        u = iota2 == fl
        fmin = jnp.where(u, nv, fmin)
        nv = jnp.where(u, big, nv)
        tiw = jnp.where(u, lax.bitwise_xor(tiw, one), tiw)


def _run_knn(new_xyz, xyz, interpret=False):
    B, N, _ = xyz.shape
    S = new_xyz.shape[1]
    xyz_pairs = xyz.reshape(B, N // 2, 2, 3)
    xyz_perm = jnp.concatenate([xyz_pairs[:, :, 0, :], xyz_pairs[:, :, 1, :]],
                               axis=1)
    xyzT = jnp.swapaxes(xyz_perm, 1, 2)
    return pl.pallas_call(
        _knn_kernel,
        grid=(B,),
        in_specs=[
            pl.BlockSpec((1, S, 3), lambda b: (b, 0, 0)),
            pl.BlockSpec((1, 3, N), lambda b: (b, 0, 0)),
        ],
        out_specs=pl.BlockSpec((1, S, _K), lambda b: (b, 0, 0)),
        out_shape=jax.ShapeDtypeStruct((B, S, _K), jnp.int32),
        interpret=interpret,
    )(-2.0 * new_xyz, xyzT)


def _group_kernel(points_hbm, idx_hbm, fps_hbm, out_hbm,
                  idx_v, idx_v2, cidx_a, cidx_b, rows_v, rows_v2, ctr_v,
                  out_v, out_v2, sem, sem_g2, sem_s1, sem_s2, sem2):
    NC = 2
    K = _K
    Q = 4
    QW = 256
    NCHUNK = QW // Q
    wid = lax.axis_index("s") * NC + lax.axis_index("c")
    base_val = (wid // 4) * 4096
    q0 = wid * QW

    pltpu.sync_copy(fps_hbm.at[pl.ds(q0, 128)], cidx_a)
    pltpu.sync_copy(fps_hbm.at[pl.ds(q0 + 128, 128)], cidx_b)

    def addbase_c(i, carry):
        sl = pl.ds(i * 16, 16)
        cidx_a[sl] = cidx_a[sl] + base_val
        cidx_b[sl] = cidx_b[sl] + base_val
        return carry

    lax.fori_loop(0, 8, addbase_c, 0)
    pltpu.async_copy(points_hbm.at[cidx_a], ctr_v.at[pl.ds(0, 128)], sem2).wait()
    pltpu.async_copy(points_hbm.at[cidx_b], ctr_v.at[pl.ds(128, 128)], sem2).wait()

    idx_b = (idx_v, idx_v2)
    rows_b = (rows_v, rows_v2)
    out_b = (out_v, out_v2)
    gs_b = (sem, sem_g2)
    ss_b = (sem_s1, sem_s2)

    def issue_gather(c, p):
        qbase = q0 + c * Q
        pltpu.sync_copy(idx_hbm.at[pl.ds(qbase * K, Q * K)], idx_b[p])

        def addbase(i, carry2):
            sl = pl.ds(i * 16, 16)
            idx_b[p][sl] = idx_b[p][sl] + base_val
            return carry2

        lax.fori_loop(0, Q * K // 16, addbase, 0)
        pltpu.make_async_copy(points_hbm.at[idx_b[p]], rows_b[p], gs_b[p]).start()

    def wait_gather(p):
        pltpu.make_async_copy(points_hbm.at[idx_b[p]], rows_b[p], gs_b[p]).wait()

    def issue_store(c, p):
        qbase = q0 + c * Q
        pltpu.make_async_copy(
            out_b[p], out_hbm.at[pl.ds(qbase * K, Q * K)], ss_b[p]).start()

    def wait_store(c, p):
        qbase = q0 + c * Q
        pltpu.make_async_copy(
            out_b[p], out_hbm.at[pl.ds(qbase * K, Q * K)], ss_b[p]).wait()

    def compute(c, p):
        def qbody(q, carry2):
            cvs = [ctr_v[c * Q + q, pl.ds(t * 16, 16)] for t in range(4)]

            def jbody(j, carry3):
                r = q * K + j
                for t in range(4):
                    g = rows_b[p][r, pl.ds(t * 16, 16)]
                    out_b[p][r, pl.ds(t * 16, 16)] = g - cvs[t]
                    out_b[p][r, pl.ds(64 + t * 16, 16)] = cvs[t]
                return carry3

            return lax.fori_loop(0, K, jbody, carry2)

        lax.fori_loop(0, Q, qbody, 0)

    issue_gather(0, 0)

    def outer_body(i, carry):
        for b in range(2):
            c = 2 * i + b

            @pl.when(c + 1 < NCHUNK)
            def _():
                issue_gather(c + 1, 1 - b)

            wait_gather(b)

            @pl.when(c >= 2)
            def _():
                wait_store(c - 2, b)

            compute(c, b)
            issue_store(c, b)
        return carry

    lax.fori_loop(0, NCHUNK // 2, outer_body, 0)
    wait_store(NCHUNK - 2, 0)
    wait_store(NCHUNK - 1, 1)


def _run_group(points, idx, fps_idx):
    B, N, D = points.shape
    S = idx.shape[1]
    K = _K
    Q = 4
    QW = 256
    points_flat = jnp.concatenate(
        [points, jnp.zeros_like(points)], axis=-1).reshape(B * N, 2 * D)
    idx_flat = idx.reshape(B * S * K)
    fps_flat = fps_idx.reshape(B * S)
    from jax.experimental.pallas import tpu_sc as plsc
    mesh = plsc.VectorSubcoreMesh(core_axis_name="c", subcore_axis_name="s")
    out = pl.kernel(
        _group_kernel,
        out_type=jax.ShapeDtypeStruct((B * S * K, 2 * D), jnp.float32),
        mesh=mesh,
        scratch_types=[
            pltpu.VMEM((Q * K,), jnp.int32),
            pltpu.VMEM((Q * K,), jnp.int32),
            pltpu.VMEM((128,), jnp.int32),
            pltpu.VMEM((128,), jnp.int32),
            pltpu.VMEM((Q * K, 2 * D), jnp.float32),
            pltpu.VMEM((Q * K, 2 * D), jnp.float32),
            pltpu.VMEM((QW, 2 * D), jnp.float32),
            pltpu.VMEM((Q * K, 2 * D), jnp.float32),
            pltpu.VMEM((Q * K, 2 * D), jnp.float32),
            pltpu.SemaphoreType.DMA,
            pltpu.SemaphoreType.DMA,
            pltpu.SemaphoreType.DMA,
            pltpu.SemaphoreType.DMA,
            pltpu.SemaphoreType.DMA,
        ],
    )(points_flat, idx_flat, fps_flat)
    return out.reshape(B, S, K, 2 * D)


def kernel(xyz, points):
    B, N, _ = xyz.shape
    D = points.shape[-1]
    fps_idx, new_xyz = _run_fps(xyz)

    idx = _run_knn(new_xyz, xyz)
    out = _run_group(points, idx, fps_idx)
    return (new_xyz, out)
